# baseline (speedup 1.0000x reference)
import functools

import jax
import jax.numpy as jnp
from jax import lax
from jax.experimental import pallas as pl
from jax.experimental.pallas import tpu as pltpu
from jax.experimental.pallas import tpu_sc as plsc

NC, NS = 2, 16
NW = NC * NS
RANK = 32
DIM = 128
NSEG = DIM // RANK
TPBLK = 512


def _tp_body(x0_ref, x1_ref, x2_ref, x3_ref, o_ref):
    eye = jnp.eye(RANK, dtype=jnp.float32)
    cols = [
        lax.dot_general(x_ref[...], eye, (((0,), (0,)), ((), ())),
                        preferred_element_type=jnp.float32)
        for x_ref in (x0_ref, x1_ref, x2_ref, x3_ref)
    ]
    o_ref[...] = jnp.concatenate(cols, axis=1)


@functools.partial(jax.jit, static_argnames=("nblk", "last3"))
def _transpose_pack(aT, *, nblk, last3):
    s_rows = nblk * TPBLK

    def mk_map(seg):
        if seg < NSEG - 1:
            return lambda i: (0, seg * nblk + i)
        return lambda i: (0, seg * nblk + jnp.minimum(i, last3))

    return pl.pallas_call(
        _tp_body,
        grid=(nblk,),
        in_specs=[pl.BlockSpec((RANK, TPBLK), mk_map(s)) for s in range(NSEG)],
        out_specs=pl.BlockSpec((TPBLK, DIM), lambda i: (i, 0)),
        out_shape=jax.ShapeDtypeStruct((s_rows, DIM), jnp.float32),
    )(aT, aT, aT, aT)


@functools.partial(jax.jit, static_argnames=("n", "chunk"))
def _sc_gather(idx_flat, table, *, n, chunk):
    per_w = n // NW
    n_chunks = per_w // chunk
    n_groups = n_chunks // 2
    mesh = plsc.VectorSubcoreMesh(core_axis_name="c", subcore_axis_name="s")

    @functools.partial(
        pl.kernel,
        out_type=jax.ShapeDtypeStruct((n, DIM), jnp.float32),
        mesh=mesh,
        scratch_types=[
            pltpu.VMEM((per_w,), jnp.int32),
            pltpu.VMEM((chunk, RANK), jnp.float32),
            pltpu.VMEM((chunk, RANK), jnp.float32),
            pltpu.SemaphoreType.DMA,
            pltpu.SemaphoreType.DMA,
            pltpu.SemaphoreType.DMA,
            pltpu.SemaphoreType.DMA,
        ],
        compiler_params=pltpu.CompilerParams(use_tc_tiling_on_sc=False),
    )
    def gather_kernel(idx_hbm, table_hbm, out_hbm, idx_v, rows0, rows1,
                      gsem0, gsem1, wsem0, wsem1):
        wid = lax.axis_index("s") * NC + lax.axis_index("c")
        base = wid * per_w
        pltpu.sync_copy(idx_hbm.at[pl.ds(base, per_w)], idx_v)

        def fire(c, rows, gsem):
            pltpu.async_copy(
                table_hbm.at[idx_v.at[pl.ds(c * chunk, chunk)]], rows, gsem)

        def gwait(rows, gsem):
            pltpu.make_async_copy(
                table_hbm.at[pl.ds(0, chunk)], rows, gsem).wait()

        def wstart(c, rows, wsem):
            pltpu.async_copy(
                rows,
                out_hbm.at[pl.ds(base + c * chunk, chunk), pl.ds(0, RANK)],
                wsem)

        def wwait(rows, wsem):
            pltpu.make_async_copy(
                rows,
                out_hbm.at[pl.ds(base, chunk), pl.ds(0, RANK)],
                wsem).wait()

        fire(0, rows0, gsem0)
        fire(1, rows1, gsem1)

        def body(g, carry):
            c = 2 * g
            gwait(rows0, gsem0)
            wstart(c, rows0, wsem0)
            gwait(rows1, gsem1)
            wstart(c + 1, rows1, wsem1)
            wwait(rows0, wsem0)
            fire(c + 2, rows0, gsem0)
            wwait(rows1, wsem1)
            fire(c + 3, rows1, gsem1)
            return carry

        lax.fori_loop(0, n_groups - 1, body, 0)

        c_last = 2 * (n_groups - 1)
        gwait(rows0, gsem0)
        wstart(c_last, rows0, wsem0)
        gwait(rows1, gsem1)
        wstart(c_last + 1, rows1, wsem1)
        wwait(rows0, wsem0)
        wwait(rows1, wsem1)

    return gather_kernel(idx_flat, table)


def _mm_body(g_ref, b_ref, o_ref):
    o_ref[...] = jnp.dot(g_ref[:, :RANK], b_ref[...],
                         preferred_element_type=jnp.float32)


@functools.partial(jax.jit, static_argnames=("blk",))
def _expand(g, b, *, blk):
    n = g.shape[0]
    return pl.pallas_call(
        _mm_body,
        grid=(n // blk,),
        in_specs=[
            pl.BlockSpec((blk, DIM), lambda i: (i, 0)),
            pl.BlockSpec((RANK, DIM), lambda i: (0, 0)),
        ],
        out_specs=pl.BlockSpec((blk, DIM), lambda i: (i, 0)),
        out_shape=jax.ShapeDtypeStruct((n, DIM), jnp.float32),
    )(g, b)


def kernel(idx, a, b):
    bsz, feat = idx.shape
    n = bsz * feat
    v = a.shape[0]
    nblk = pl.cdiv(pl.cdiv(v, NSEG), TPBLK)
    s_rows = nblk * TPBLK
    last3 = (v - (NSEG - 1) * s_rows) // TPBLK

    aT = jnp.transpose(a)
    idx_t = jnp.transpose(idx).reshape(n)
    seg = idx_t // s_rows
    idx_p = NSEG * (idx_t - seg * s_rows) + seg

    packed = _transpose_pack(aT, nblk=nblk, last3=last3)
    a_dense = packed.reshape(s_rows * DIM).reshape(NSEG * s_rows, RANK)
    g = _sc_gather(idx_p, a_dense, n=n, chunk=1664)
    out = _expand(g, b, blk=8192)
    return jnp.transpose(out.reshape(feat, bsz, DIM), (1, 0, 2))


# speedup vs baseline: 1.3728x; 1.3728x over previous
import functools

import jax
import jax.numpy as jnp
from jax import lax
from jax.experimental import pallas as pl
from jax.experimental.pallas import tpu as pltpu
from jax.experimental.pallas import tpu_sc as plsc

NC, NS = 2, 16
NW = NC * NS
RANK = 32
DIM = 128
NSEG = DIM // RANK
TPBLK = 4096


def _tp_body(x0_ref, x1_ref, x2_ref, x3_ref, o_ref):
    cols = [x_ref[...].T for x_ref in (x0_ref, x1_ref, x2_ref, x3_ref)]
    o_ref[...] = jnp.concatenate(cols, axis=1)


@functools.partial(jax.jit, static_argnames=("nblk", "last3"))
def _transpose_pack(aT, *, nblk, last3):
    s_rows = nblk * TPBLK

    def mk_map(seg):
        if seg < NSEG - 1:
            return lambda i: (0, seg * nblk + i)
        return lambda i: (0, seg * nblk + jnp.minimum(i, last3))

    return pl.pallas_call(
        _tp_body,
        grid=(nblk,),
        in_specs=[pl.BlockSpec((RANK, TPBLK), mk_map(s)) for s in range(NSEG)],
        out_specs=pl.BlockSpec((TPBLK, DIM), lambda i: (i, 0)),
        out_shape=jax.ShapeDtypeStruct((s_rows, DIM), jnp.float32),
    )(aT, aT, aT, aT)


@functools.partial(jax.jit, static_argnames=("n", "chunk"))
def _sc_gather(idx_flat, table, *, n, chunk):
    per_w = n // NW
    n_chunks = per_w // chunk
    n_groups = n_chunks // 2
    mesh = plsc.VectorSubcoreMesh(core_axis_name="c", subcore_axis_name="s")

    @functools.partial(
        pl.kernel,
        out_type=jax.ShapeDtypeStruct((n, DIM), jnp.float32),
        mesh=mesh,
        scratch_types=[
            pltpu.VMEM((per_w,), jnp.int32),
            pltpu.VMEM((chunk, RANK), jnp.float32),
            pltpu.VMEM((chunk, RANK), jnp.float32),
            pltpu.SemaphoreType.DMA,
            pltpu.SemaphoreType.DMA,
            pltpu.SemaphoreType.DMA,
            pltpu.SemaphoreType.DMA,
        ],
        compiler_params=pltpu.CompilerParams(use_tc_tiling_on_sc=False),
    )
    def gather_kernel(idx_hbm, table_hbm, out_hbm, idx_v, rows0, rows1,
                      gsem0, gsem1, wsem0, wsem1):
        wid = lax.axis_index("s") * NC + lax.axis_index("c")
        base = wid * per_w
        pltpu.sync_copy(idx_hbm.at[pl.ds(base, per_w)], idx_v)

        def fire(c, rows, gsem):
            pltpu.async_copy(
                table_hbm.at[idx_v.at[pl.ds(c * chunk, chunk)]], rows, gsem)

        def gwait(rows, gsem):
            pltpu.make_async_copy(
                table_hbm.at[pl.ds(0, chunk)], rows, gsem).wait()

        def wstart(c, rows, wsem):
            pltpu.async_copy(
                rows,
                out_hbm.at[pl.ds(base + c * chunk, chunk), pl.ds(0, RANK)],
                wsem)

        def wwait(rows, wsem):
            pltpu.make_async_copy(
                rows,
                out_hbm.at[pl.ds(base, chunk), pl.ds(0, RANK)],
                wsem).wait()

        fire(0, rows0, gsem0)
        fire(1, rows1, gsem1)

        def body(g, carry):
            c = 2 * g
            gwait(rows0, gsem0)
            wstart(c, rows0, wsem0)
            gwait(rows1, gsem1)
            wstart(c + 1, rows1, wsem1)
            wwait(rows0, wsem0)
            fire(c + 2, rows0, gsem0)
            wwait(rows1, wsem1)
            fire(c + 3, rows1, gsem1)
            return carry

        lax.fori_loop(0, n_groups - 1, body, 0)

        c_last = 2 * (n_groups - 1)
        gwait(rows0, gsem0)
        wstart(c_last, rows0, wsem0)
        gwait(rows1, gsem1)
        wstart(c_last + 1, rows1, wsem1)
        wwait(rows0, wsem0)
        wwait(rows1, wsem1)

    return gather_kernel(idx_flat, table)


def _mm_body(g_ref, b_ref, o_ref):
    o_ref[...] = jnp.dot(g_ref[:, :RANK], b_ref[...],
                         preferred_element_type=jnp.float32)


@functools.partial(jax.jit, static_argnames=("blk",))
def _expand(g, b, *, blk):
    n = g.shape[0]
    return pl.pallas_call(
        _mm_body,
        grid=(n // blk,),
        in_specs=[
            pl.BlockSpec((blk, DIM), lambda i: (i, 0)),
            pl.BlockSpec((RANK, DIM), lambda i: (0, 0)),
        ],
        out_specs=pl.BlockSpec((blk, DIM), lambda i: (i, 0)),
        out_shape=jax.ShapeDtypeStruct((n, DIM), jnp.float32),
    )(g, b)


def kernel(idx, a, b):
    bsz, feat = idx.shape
    n = bsz * feat
    v = a.shape[0]
    nblk = pl.cdiv(pl.cdiv(v, NSEG), TPBLK)
    s_rows = nblk * TPBLK
    last3 = (v - (NSEG - 1) * s_rows) // TPBLK

    aT = jnp.transpose(a)
    idx_t = jnp.transpose(idx).reshape(n)
    seg = idx_t // s_rows
    idx_p = NSEG * (idx_t - seg * s_rows) + seg

    packed = _transpose_pack(aT, nblk=nblk, last3=last3)
    a_dense = packed.reshape(s_rows * DIM).reshape(NSEG * s_rows, RANK)
    g = _sc_gather(idx_p, a_dense, n=n, chunk=1664)
    out = _expand(g, b, blk=8192)
    return jnp.transpose(out.reshape(feat, bsz, DIM), (1, 0, 2))


# speedup vs baseline: 2.0469x; 1.4911x over previous
import functools

import jax
import jax.numpy as jnp
from jax import lax
from jax.experimental import pallas as pl
from jax.experimental.pallas import tpu as pltpu
from jax.experimental.pallas import tpu_sc as plsc

NC, NS = 2, 16
NW = NC * NS
RANK = 32
DIM = 128
NSEG = DIM // RANK
TPBLK = 4096


def _tp_body(x0_ref, x1_ref, x2_ref, x3_ref, o_ref):
    stacked = jnp.concatenate(
        [x0_ref[...], x1_ref[...], x2_ref[...], x3_ref[...]], axis=0)
    o_ref[...] = stacked.T


@functools.partial(jax.jit, static_argnames=("nblk", "last3"))
def _transpose_pack(aT, *, nblk, last3):
    s_rows = nblk * TPBLK

    def mk_map(seg):
        if seg < NSEG - 1:
            return lambda i: (0, seg * nblk + i)
        return lambda i: (0, seg * nblk + jnp.minimum(i, last3))

    return pl.pallas_call(
        _tp_body,
        grid=(nblk,),
        in_specs=[pl.BlockSpec((RANK, TPBLK), mk_map(s)) for s in range(NSEG)],
        out_specs=pl.BlockSpec((TPBLK, DIM), lambda i: (i, 0)),
        out_shape=jax.ShapeDtypeStruct((s_rows, DIM), jnp.float32),
    )(aT, aT, aT, aT)


@functools.partial(jax.jit, static_argnames=("n", "chunk"))
def _sc_gather(idx_flat, table, *, n, chunk):
    per_w = n // NW
    n_chunks = per_w // chunk
    n_groups = n_chunks // 2
    mesh = plsc.VectorSubcoreMesh(core_axis_name="c", subcore_axis_name="s")

    @functools.partial(
        pl.kernel,
        out_type=jax.ShapeDtypeStruct((n, DIM), jnp.float32),
        mesh=mesh,
        scratch_types=[
            pltpu.VMEM((per_w,), jnp.int32),
            pltpu.VMEM((chunk, RANK), jnp.float32),
            pltpu.VMEM((chunk, RANK), jnp.float32),
            pltpu.SemaphoreType.DMA,
            pltpu.SemaphoreType.DMA,
            pltpu.SemaphoreType.DMA,
            pltpu.SemaphoreType.DMA,
        ],
        compiler_params=pltpu.CompilerParams(use_tc_tiling_on_sc=False),
    )
    def gather_kernel(idx_hbm, table_hbm, out_hbm, idx_v, rows0, rows1,
                      gsem0, gsem1, wsem0, wsem1):
        wid = lax.axis_index("s") * NC + lax.axis_index("c")
        base = wid * per_w
        pltpu.sync_copy(idx_hbm.at[pl.ds(base, per_w)], idx_v)

        def fire(c, rows, gsem):
            pltpu.async_copy(
                table_hbm.at[idx_v.at[pl.ds(c * chunk, chunk)]], rows, gsem)

        def gwait(rows, gsem):
            pltpu.make_async_copy(
                table_hbm.at[pl.ds(0, chunk)], rows, gsem).wait()

        def wstart(c, rows, wsem):
            pltpu.async_copy(
                rows,
                out_hbm.at[pl.ds(base + c * chunk, chunk), pl.ds(0, RANK)],
                wsem)

        def wwait(rows, wsem):
            pltpu.make_async_copy(
                rows,
                out_hbm.at[pl.ds(base, chunk), pl.ds(0, RANK)],
                wsem).wait()

        fire(0, rows0, gsem0)
        fire(1, rows1, gsem1)

        def body(g, carry):
            c = 2 * g
            gwait(rows0, gsem0)
            wstart(c, rows0, wsem0)
            gwait(rows1, gsem1)
            wstart(c + 1, rows1, wsem1)
            wwait(rows0, wsem0)
            fire(c + 2, rows0, gsem0)
            wwait(rows1, wsem1)
            fire(c + 3, rows1, gsem1)
            return carry

        lax.fori_loop(0, n_groups - 1, body, 0)

        c_last = 2 * (n_groups - 1)
        gwait(rows0, gsem0)
        wstart(c_last, rows0, wsem0)
        gwait(rows1, gsem1)
        wstart(c_last + 1, rows1, wsem1)
        wwait(rows0, wsem0)
        wwait(rows1, wsem1)

    return gather_kernel(idx_flat, table)


def _mm_body(g_ref, b_ref, o_ref):
    o_ref[...] = jnp.dot(g_ref[:, :RANK], b_ref[...],
                         preferred_element_type=jnp.float32)


@functools.partial(jax.jit, static_argnames=("blk",))
def _expand(g, b, *, blk):
    n = g.shape[0]
    return pl.pallas_call(
        _mm_body,
        grid=(n // blk,),
        in_specs=[
            pl.BlockSpec((blk, DIM), lambda i: (i, 0)),
            pl.BlockSpec((RANK, DIM), lambda i: (0, 0)),
        ],
        out_specs=pl.BlockSpec((blk, DIM), lambda i: (i, 0)),
        out_shape=jax.ShapeDtypeStruct((n, DIM), jnp.float32),
    )(g, b)


def kernel(idx, a, b):
    bsz, feat = idx.shape
    n = bsz * feat
    v = a.shape[0]
    nblk = pl.cdiv(pl.cdiv(v, NSEG), TPBLK)
    s_rows = nblk * TPBLK
    last3 = (v - (NSEG - 1) * s_rows) // TPBLK

    aT = jnp.transpose(a)
    idx_t = jnp.transpose(idx).reshape(n)
    seg = idx_t // s_rows
    idx_p = NSEG * (idx_t - seg * s_rows) + seg

    packed = _transpose_pack(aT, nblk=nblk, last3=last3)
    a_dense = packed.reshape(s_rows * DIM).reshape(NSEG * s_rows, RANK)
    g = _sc_gather(idx_p, a_dense, n=n, chunk=1664)
    out = _expand(g, b, blk=8192)
    return jnp.transpose(out.reshape(feat, bsz, DIM), (1, 0, 2))
